# baseline (speedup 1.0000x reference)
import dataclasses
import functools

import jax
import jax.numpy as jnp
from jax import lax
from jax.experimental import pallas as pl
from jax.experimental.pallas import tpu as pltpu
from jax.experimental.pallas import tpu_sc as plsc

N_NODES = 10000
N_PAD = 10112
IN_CH = 128
OUT_CH = 128
NUM_T = 7
SCALING = 100.0
WIDTH = 144
NC = 2
NS = 16
NW = NC * NS
CHUNK = 128
ROW_BLK = 1000


def _scaled_table(x_src, weight, relation_weight):
    def body(rw_ref, x_ref, w_ref, out_ref, acc_ref):
        t = pl.program_id(1)

        @pl.when(t == 0)
        def _():
            acc_ref[...] = jnp.dot(
                x_ref[...], w_ref[...], preferred_element_type=jnp.float32
            )

        s = rw_ref[t] * SCALING
        s = jnp.where(s >= 0.0, s, 0.01 * s)
        out_ref[:, :OUT_CH] = acc_ref[...] * s
        out_ref[:, OUT_CH:] = jnp.full((ROW_BLK, WIDTH - OUT_CH), s, jnp.float32)

    n_blk = N_NODES // ROW_BLK
    return pl.pallas_call(
        body,
        grid=(n_blk, NUM_T),
        in_specs=[
            pl.BlockSpec(memory_space=pltpu.SMEM),
            pl.BlockSpec((ROW_BLK, IN_CH), lambda i, t: (i, 0)),
            pl.BlockSpec((IN_CH, OUT_CH), lambda i, t: (0, 0)),
        ],
        out_specs=pl.BlockSpec((ROW_BLK, WIDTH), lambda i, t: (t * n_blk + i, 0)),
        out_shape=jax.ShapeDtypeStruct((NUM_T * N_NODES, WIDTH), jnp.float32),
        scratch_shapes=[pltpu.VMEM((ROW_BLK, OUT_CH), jnp.float32)],
    )(relation_weight, x_src, weight)


CHUNKS_PER_TILE = 80


def _sc_aggregate(table, packed, zeros):
    rows_per_sub = N_PAD // NS
    nchunks = CHUNKS_PER_TILE
    mesh = plsc.VectorSubcoreMesh(core_axis_name="c", subcore_axis_name="s")

    @functools.partial(
        pl.kernel,
        mesh=mesh,
        out_type=jax.ShapeDtypeStruct((NC, N_PAD, WIDTH), jnp.float32),
        scratch_types=[
            [pltpu.VMEM((3, CHUNK), jnp.int32) for _ in range(4)],
            [pltpu.VMEM((CHUNK,), jnp.int32) for _ in range(2)],
            [pltpu.VMEM((CHUNK, WIDTH), jnp.float32) for _ in range(2)],
            pltpu.VMEM_SHARED((N_PAD, WIDTH), jnp.float32),
            [pltpu.SemaphoreType.DMA for _ in range(4)],
            [pltpu.SemaphoreType.DMA for _ in range(2)],
        ],
        compiler_params=dataclasses.replace(
            pltpu.CompilerParams(), use_tc_tiling_on_sc=False
        ),
    )
    def k(table_hbm, idx_hbm, zeros_hbm, out_hbm,
          idx_r, gidx, rows, acc, isem, gsem):
        cid = lax.axis_index("c")
        sid = lax.axis_index("s")
        wid = sid * NC + cid

        sub_slc = pl.ds(sid * rows_per_sub, rows_per_sub)
        pltpu.sync_copy(zeros_hbm.at[sub_slc], acc.at[sub_slc])
        plsc.subcore_barrier()

        def idx_load(j, q):
            pltpu.async_copy(idx_hbm.at[wid, j], idx_r[q], isem[q])

        def idx_wait(q):
            pltpu.make_async_copy(idx_hbm.at[wid, 0], idx_r[q], isem[q]).wait()

        def compute_gidx(q, x):
            @pl.loop(0, CHUNK // 16)
            def _(kk):
                sl = pl.ds(kk * 16, 16)
                gidx[x][sl] = idx_r[q][1, sl] * N_NODES + idx_r[q][0, sl]

        def gather_start(x):
            pltpu.async_copy(table_hbm.at[gidx[x]], rows[x], gsem[x])

        def gather_wait(x):
            pltpu.make_async_copy(table_hbm.at[gidx[x]], rows[x], gsem[x]).wait()

        def scatter(q, x):
            pltpu.sync_copy(rows[x], acc.at[idx_r[q].at[2]], add=True)

        for q in range(4):
            idx_load(q, q)
        idx_wait(0)
        compute_gidx(0, 0)
        gather_start(0)

        @pl.loop(0, nchunks // 4)
        def _(m):
            j0 = 4 * m
            for r in range(4):
                j = j0 + r
                x, xn, q, qn = r % 2, (r + 1) % 2, r, (r + 1) % 4

                @pl.when(j + 1 < nchunks)
                def _():
                    idx_wait(qn)
                    compute_gidx(qn, xn)
                    gather_start(xn)

                gather_wait(x)
                scatter(q, x)

                @pl.when(j + 4 < nchunks)
                def _():
                    idx_load(j + 4, q)

        plsc.subcore_barrier()
        pltpu.sync_copy(acc.at[sub_slc], out_hbm.at[cid, sub_slc])

    return k(table, packed, zeros)


def _finalize(partial, bias):
    def body(p_ref, b_ref, o_ref):
        a = p_ref[0] + p_ref[1]
        deg = a[:, OUT_CH:OUT_CH + 1]
        inv = jnp.where(deg != 0.0, 1.0 / deg, 0.0)
        o_ref[...] = a[:, :OUT_CH] * inv + b_ref[...]

    n_blk = N_NODES // ROW_BLK
    return pl.pallas_call(
        body,
        grid=(n_blk,),
        in_specs=[
            pl.BlockSpec((NC, ROW_BLK, WIDTH), lambda i: (0, i, 0)),
            pl.BlockSpec((1, OUT_CH), lambda i: (0, 0)),
        ],
        out_specs=pl.BlockSpec((ROW_BLK, OUT_CH), lambda i: (i, 0)),
        out_shape=jax.ShapeDtypeStruct((N_NODES, OUT_CH), jnp.float32),
    )(partial, bias)


def kernel(x_src, x_target, edge_index, edge_type, target_node_type,
           weight, bias, relation_weight):
    n_edges = edge_index.shape[1]
    e_pad = NW * CHUNKS_PER_TILE * CHUNK
    pad = e_pad - n_edges
    assert pad >= 0
    row = edge_index[0].astype(jnp.int32)
    col = edge_index[1].astype(jnp.int32)
    ty = edge_type.astype(jnp.int32)
    row = jnp.concatenate([row, jnp.zeros((pad,), jnp.int32)])
    ty = jnp.concatenate([ty, jnp.zeros((pad,), jnp.int32)])
    col = jnp.concatenate([col, jnp.full((pad,), N_NODES, jnp.int32)])
    packed = jnp.stack([row, ty, col])
    packed = packed.reshape(3, NW, CHUNKS_PER_TILE, CHUNK).transpose(1, 2, 0, 3)
    table = _scaled_table(x_src, weight, relation_weight.astype(jnp.float32))
    zeros = jnp.zeros((N_PAD, WIDTH), jnp.float32)
    partial = _sc_aggregate(table, packed, zeros)
    return _finalize(partial, bias.reshape(1, OUT_CH))
